# baseline (speedup 1.0000x reference)
import functools

import jax
import jax.numpy as jnp
from jax import lax
from jax.experimental import pallas as pl
from jax.experimental.pallas import tpu as pltpu
from jax.experimental.pallas import tpu_sc as plsc

N_NODES = 10000
D = 128
N_EDGES = 320000

NC = 2
NS = 16
NW = NC * NS
CHUNK = 80
CHUNKS_PER_TILE = 125
EDGES_PER_TILE = CHUNK * CHUNKS_PER_TILE
N_PAD = 10240
ROWS_PER_TILE = N_PAD // NS


def _make_sc_messages():
    mesh = plsc.VectorSubcoreMesh(core_axis_name="c", subcore_axis_name="s")

    @functools.partial(
        pl.kernel,
        mesh=mesh,
        out_type=jax.ShapeDtypeStruct((NC * N_PAD, D), jnp.float32),
        scratch_types=[
            [pltpu.VMEM((CHUNK,), jnp.int32) for _ in range(4)],
            [pltpu.VMEM((CHUNK,), jnp.int32) for _ in range(4)],
            [pltpu.VMEM((CHUNK,), jnp.float32) for _ in range(4)],
            [pltpu.VMEM((CHUNK, D), jnp.float32) for _ in range(2)],
            [pltpu.VMEM((CHUNK, D), jnp.float32) for _ in range(2)],
            pltpu.VMEM_SHARED((N_PAD, D), jnp.float32),
            [pltpu.SemaphoreType.DMA for _ in range(2)],
            [pltpu.SemaphoreType.DMA for _ in range(2)],
            [pltpu.SemaphoreType.DMA for _ in range(4)],
        ],
    )
    def body(x_hbm, src_hbm, dst_hbm, e_hbm, out_hbm,
             idx_s, idx_d, ev, rows_s, rows_d, acc, gsem, ssem, isem):
        cid = lax.axis_index("c")
        sid = lax.axis_index("s")
        wid = cid * NS + sid

        zero = jnp.zeros((16,), jnp.float32)

        def zrow(i, carry):
            for r in range(D // 16):
                rows_s[0][i, pl.ds(r * 16, 16)] = zero
            return carry

        lax.fori_loop(0, CHUNK, zrow, 0)
        r0 = sid * ROWS_PER_TILE
        for t in range(ROWS_PER_TILE // CHUNK):
            pltpu.sync_copy(rows_s[0], acc.at[pl.ds(r0 + t * CHUNK, CHUNK)])
        plsc.subcore_barrier()

        base0 = wid * EDGES_PER_TILE

        def copy_idx(c, m):
            base = base0 + c * CHUNK
            pltpu.async_copy(src_hbm.at[pl.ds(base, CHUNK)], idx_s[m], isem[m])
            pltpu.async_copy(dst_hbm.at[pl.ds(base, CHUNK)], idx_d[m], isem[m])
            pltpu.async_copy(e_hbm.at[pl.ds(base, CHUNK)], ev[m], isem[m])

        def wait_idx(m):
            pltpu.make_async_copy(src_hbm.at[pl.ds(0, CHUNK)], idx_s[m], isem[m]).wait()
            pltpu.make_async_copy(dst_hbm.at[pl.ds(0, CHUNK)], idx_d[m], isem[m]).wait()
            pltpu.make_async_copy(e_hbm.at[pl.ds(0, CHUNK)], ev[m], isem[m]).wait()

        def issue_gather(m, p):
            pltpu.async_copy(x_hbm.at[idx_s[m]], rows_s[p], gsem[p])
            pltpu.async_copy(x_hbm.at[idx_d[m]], rows_d[p], gsem[p])

        def wait_gather(m, p):
            pltpu.make_async_copy(x_hbm.at[idx_s[m]], rows_s[p], gsem[p]).wait()
            pltpu.make_async_copy(x_hbm.at[idx_d[m]], rows_d[p], gsem[p]).wait()

        def scale(m, p):
            bs, bd = rows_s[p], rows_d[p]

            def sgroup(g, inner):
                ev16 = ev[m][pl.ds(g * 16, 16)]
                i0 = g * 16
                for j in range(16):
                    eb = jnp.full((16,), ev16[j], jnp.float32)
                    for r in range(D // 16):
                        sl = pl.ds(r * 16, 16)
                        bs[i0 + j, sl] = bs[i0 + j, sl] * eb
                        bd[i0 + j, sl] = bd[i0 + j, sl] * eb
                return inner

            lax.fori_loop(0, CHUNK // 16, sgroup, 0)

        def issue_scatter(m, p):
            pltpu.async_copy(rows_s[p], acc.at[idx_d[m]], ssem[p], add=True)
            pltpu.async_copy(rows_d[p], acc.at[idx_s[m]], ssem[p], add=True)

        def wait_scatter(m, p):
            pltpu.make_async_copy(rows_s[p], acc.at[idx_d[m]], ssem[p]).wait()
            pltpu.make_async_copy(rows_d[p], acc.at[idx_s[m]], ssem[p]).wait()

        def steady(c, m):
            p = m % 2
            q = 1 - p
            mg = (m + 1) % 4
            mc = (m + 2) % 4
            mq = (m - 1) % 4
            wait_gather(m, p)
            scale(m, p)
            issue_scatter(m, p)
            wait_scatter(mq, q)
            wait_idx(mg)
            issue_gather(mg, q)
            copy_idx(c + 2, mc)

        copy_idx(0, 0)
        wait_idx(0)
        issue_gather(0, 0)
        copy_idx(1, 1)
        wait_gather(0, 0)
        scale(0, 0)
        issue_scatter(0, 0)
        wait_idx(1)
        issue_gather(1, 1)
        copy_idx(2, 2)
        steady(1, 1)

        def four(g, carry):
            c = 2 + 4 * g
            steady(c + 0, 2)
            steady(c + 1, 3)
            steady(c + 2, 0)
            steady(c + 3, 1)
            return carry

        lax.fori_loop(0, (CHUNKS_PER_TILE - 5) // 4, four, 0)

        steady(CHUNKS_PER_TILE - 3, 2)
        wait_gather(3, 1)
        scale(3, 1)
        issue_scatter(3, 1)
        wait_scatter(2, 0)
        wait_idx(0)
        issue_gather(0, 0)
        wait_gather(0, 0)
        scale(0, 0)
        issue_scatter(0, 0)
        wait_scatter(3, 1)
        wait_scatter(0, 0)

        plsc.subcore_barrier()
        out_base = cid * N_PAD + r0
        pltpu.sync_copy(acc.at[pl.ds(r0, ROWS_PER_TILE)],
                        out_hbm.at[pl.ds(out_base, ROWS_PER_TILE)])

    return body


_SC_CACHE = []


def _sc_messages():
    if not _SC_CACHE:
        _SC_CACHE.append(_make_sc_messages())
    return _SC_CACHE[0]


_R = 1000


def _mlp_body(mi_ref, x_ref, w1_ref, b1_ref, g1_ref, beta1_ref, w2_ref,
              b2_ref, out_ref):
    mi = mi_ref[0] + mi_ref[1]
    h = (
        jnp.dot(mi, w1_ref[:D, :], preferred_element_type=jnp.float32,
                precision=lax.Precision.HIGHEST)
        + jnp.dot(x_ref[...], w1_ref[D:, :], preferred_element_type=jnp.float32,
                  precision=lax.Precision.HIGHEST)
        + b1_ref[...]
    )
    mean = jnp.mean(h, axis=1, keepdims=True)
    var = jnp.mean((h - mean) ** 2, axis=1, keepdims=True)
    h = (h - mean) * lax.rsqrt(var + 1e-5) * g1_ref[...] + beta1_ref[...]
    h = jnp.tanh(h)
    out_ref[...] = (
        jnp.dot(h, w2_ref[...], preferred_element_type=jnp.float32,
                precision=lax.Precision.HIGHEST)
        + b2_ref[...]
    )


def _mlp(mi2, x, w1, b1, g1, beta1, w2, b2):
    grid = (N_NODES // _R,)
    vspec = pl.BlockSpec((1, D), lambda i: (0, 0))
    return pl.pallas_call(
        _mlp_body,
        grid=grid,
        in_specs=[
            pl.BlockSpec((2, _R, D), lambda i: (0, i, 0)),
            pl.BlockSpec((_R, D), lambda i: (i, 0)),
            pl.BlockSpec((2 * D, D), lambda i: (0, 0)),
            vspec,
            vspec,
            vspec,
            pl.BlockSpec((D, D), lambda i: (0, 0)),
            vspec,
        ],
        out_specs=pl.BlockSpec((_R, D), lambda i: (i, 0)),
        out_shape=jax.ShapeDtypeStruct((N_NODES, D), jnp.float32),
    )(mi2, x, w1, b1.reshape(1, D), g1.reshape(1, D), beta1.reshape(1, D),
      w2, b2.reshape(1, D))


def kernel(x, e, edge_index, W1, b1, g1, beta1, W2, b2):
    src = edge_index[0].astype(jnp.int32)
    dst = edge_index[1].astype(jnp.int32)
    partials = _sc_messages()(x, src, dst, e)
    mi2 = partials.reshape(2, N_PAD, D)
    return _mlp(mi2, x, W1, b1, g1, beta1, W2, b2)
